# baseline (speedup 1.0000x reference)
import functools

import jax
import jax.numpy as jnp
from jax import lax
from jax.experimental import pallas as pl
from jax.experimental.pallas import tpu as pltpu
from jax.experimental.pallas import tpu_sc as plsc

DIM = 4
NUM_WORKERS = 32
LANES = 128
SUB = 8
TILE = DIM * LANES
NBUF = 3


@functools.lru_cache(maxsize=None)
def _emb_fn(n_batch, n_seq, vocab):
    n_bt = n_batch // LANES
    n_st = n_seq // SUB
    n_units = n_seq * DIM_Q
    units_per_w = n_units // NUM_WORKERS
    nt_per_u = n_bt // DIM_Q
    chunk_ids = nt_per_u * LANES
    mesh = plsc.VectorSubcoreMesh(core_axis_name="c", subcore_axis_name="s")

    @functools.partial(
        pl.kernel,
        mesh=mesh,
        out_type=jax.ShapeDtypeStruct((n_batch * n_seq * DIM,), jnp.float32),
        scratch_types=[
            pltpu.VMEM((DIM * vocab * 16,), jnp.float32),
            pltpu.VMEM((NBUF, nt_per_u, LANES), jnp.int32),
            pltpu.VMEM((NBUF * chunk_ids * DIM,), jnp.float32),
            pltpu.SemaphoreType.DMA((NBUF,)),
            pltpu.SemaphoreType.DMA((NBUF,)),
        ],
        compiler_params=pltpu.CompilerParams(needs_layout_passes=False),
    )
    def emb(ids_hbm, table_hbm, out_hbm, table_v, ids_v, out_v,
            ids_sem, out_sem):
        wid = lax.axis_index("s") * 2 + lax.axis_index("c")
        pltpu.sync_copy(table_hbm, table_v)
        u0 = wid * units_per_w
        lane_d = [
            lax.iota(jnp.int32, 16) + d * (vocab * 16) for d in range(DIM)
        ]

        def ids_copy(u, b):
            s = u // DIM_Q
            q = u % DIM_Q
            return pltpu.make_async_copy(
                ids_hbm.at[s // SUB, pl.ds(q * nt_per_u, nt_per_u), s % SUB, :],
                ids_v.at[b],
                ids_sem.at[b],
            )

        def out_copy(u, b):
            return pltpu.make_async_copy(
                out_v.at[pl.ds(b * chunk_ids * DIM, chunk_ids * DIM)],
                out_hbm.at[pl.ds(u * chunk_ids * DIM, chunk_ids * DIM)],
                out_sem.at[b],
            )

        def compute(b):
            @plsc.parallel_loop(0, chunk_ids // 16, unroll=8)
            def g_body(g):
                idv = ids_v[b, g // 8, pl.ds((g % 8) * 16, 16)]
                t16 = idv * 16
                ob = (g // 8) * TILE + (g % 8) * 16
                for d in range(DIM):
                    vals = plsc.load_gather(table_v, [t16 + lane_d[d]])
                    out_v[pl.ds(b * chunk_ids * DIM + ob + d * LANES, 16)] = (
                        vals
                    )

        def unit_body(k, carry):
            u = u0 + k
            b = k % NBUF
            out_copy(u - NBUF, b).wait()
            ids_copy(u, b).wait()
            compute(b)
            out_copy(u, b).start()

            @pl.when(k + NBUF < units_per_w)
            def _():
                ids_copy(u + NBUF, b).start()

            return carry

        for k in range(NBUF):
            ids_copy(u0 + k, k).start()
        for k in range(NBUF):
            ids_copy(u0 + k, k).wait()
            compute(k)
            out_copy(u0 + k, k).start()
            ids_copy(u0 + k + NBUF, k).start()
        lax.fori_loop(NBUF, units_per_w, unit_body, 0)
        for k in range(units_per_w - NBUF, units_per_w):
            out_copy(u0 + k, k % NBUF).wait()

    return emb


DIM_Q = 8


def kernel(input_ids, table):
    n_batch, n_seq = input_ids.shape
    ids4 = input_ids.astype(jnp.int32).reshape(
        n_batch // LANES, LANES, n_seq // SUB, SUB
    )
    ids_raw = jnp.transpose(ids4, (2, 0, 3, 1))
    table_rep = jnp.broadcast_to(
        table.T[:, :, None], (table.shape[1], table.shape[0], 16)
    ).reshape(-1)
    out_flat = _emb_fn(n_batch, n_seq, table.shape[0])(ids_raw, table_rep)
    out4 = out_flat.reshape(n_seq, n_batch // LANES, DIM, LANES)
    return jnp.transpose(out4, (1, 3, 0, 2)).reshape(n_batch, n_seq, DIM)


# speedup vs baseline: 1.0588x; 1.0588x over previous
import functools

import jax
import jax.numpy as jnp
from jax import lax
from jax.experimental import pallas as pl
from jax.experimental.pallas import tpu as pltpu
from jax.experimental.pallas import tpu_sc as plsc

DIM = 4
NUM_WORKERS = 32
LANES = 128
SUB = 8
TILE = DIM * LANES
NBUF = 3


@functools.lru_cache(maxsize=None)
def _emb_fn(n_batch, n_seq, vocab):
    n_bt = n_batch // LANES
    n_st = n_seq // SUB
    n_units = n_seq * DIM_Q
    units_per_w = n_units // NUM_WORKERS
    nt_per_u = n_bt // DIM_Q
    chunk_ids = nt_per_u * LANES
    mesh = plsc.VectorSubcoreMesh(core_axis_name="c", subcore_axis_name="s")

    @functools.partial(
        pl.kernel,
        mesh=mesh,
        out_type=jax.ShapeDtypeStruct((n_batch * n_seq * DIM,), jnp.float32),
        scratch_types=[
            pltpu.VMEM((DIM * vocab * 16,), jnp.float32),
            pltpu.VMEM((NBUF, nt_per_u, LANES), jnp.int32),
            pltpu.VMEM((NBUF * chunk_ids * DIM,), jnp.float32),
            pltpu.SemaphoreType.DMA((NBUF,)),
            pltpu.SemaphoreType.DMA((NBUF,)),
        ],
        compiler_params=pltpu.CompilerParams(needs_layout_passes=False),
    )
    def emb(ids_hbm, table_hbm, out_hbm, table_v, ids_v, out_v,
            ids_sem, out_sem):
        wid = lax.axis_index("s") * 2 + lax.axis_index("c")
        pltpu.sync_copy(table_hbm, table_v)
        u0 = wid * units_per_w
        lane_d = [
            lax.iota(jnp.int32, 16) + d * (vocab * 16) for d in range(DIM)
        ]

        def ids_copy(u, b):
            s = u // DIM_Q
            q = u % DIM_Q
            return pltpu.make_async_copy(
                ids_hbm.at[s // SUB, pl.ds(q * nt_per_u, nt_per_u), s % SUB, :],
                ids_v.at[b],
                ids_sem.at[b],
            )

        def out_copy(u, b):
            return pltpu.make_async_copy(
                out_v.at[pl.ds(b * chunk_ids * DIM, chunk_ids * DIM)],
                out_hbm.at[pl.ds(u * chunk_ids * DIM, chunk_ids * DIM)],
                out_sem.at[b],
            )

        def compute(b):
            @plsc.parallel_loop(0, chunk_ids // 16, unroll=8)
            def g_body(g):
                idv = ids_v[b, g // 8, pl.ds((g % 8) * 16, 16)]
                t16 = idv * 16
                ob = (g // 8) * TILE + (g % 8) * 16
                for d in range(DIM):
                    vals = plsc.load_gather(table_v, [t16 + lane_d[d]])
                    out_v[pl.ds(b * chunk_ids * DIM + ob + d * LANES, 16)] = (
                        vals
                    )

        def unit_body(k, carry):
            u = u0 + k
            b = k % NBUF
            out_copy(u - NBUF, b).wait()
            ids_copy(u, b).wait()
            compute(b)
            out_copy(u, b).start()

            @pl.when(k + NBUF < units_per_w)
            def _():
                ids_copy(u + NBUF, b).start()

            return carry

        for k in range(NBUF):
            ids_copy(u0 + k, k).start()
        for k in range(NBUF):
            ids_copy(u0 + k, k).wait()
            compute(k)
            out_copy(u0 + k, k).start()
            ids_copy(u0 + k + NBUF, k).start()
        lax.fori_loop(NBUF, units_per_w, unit_body, 0)
        for k in range(units_per_w - NBUF, units_per_w):
            out_copy(u0 + k, k % NBUF).wait()

    return emb


DIM_Q = 4


def kernel(input_ids, table):
    n_batch, n_seq = input_ids.shape
    ids4 = input_ids.astype(jnp.int32).reshape(
        n_batch // LANES, LANES, n_seq // SUB, SUB
    )
    ids_raw = jnp.transpose(ids4, (2, 0, 3, 1))
    table_rep = jnp.broadcast_to(
        table.T[:, :, None], (table.shape[1], table.shape[0], 16)
    ).reshape(-1)
    out_flat = _emb_fn(n_batch, n_seq, table.shape[0])(ids_raw, table_rep)
    out4 = out_flat.reshape(n_seq, n_batch // LANES, DIM, LANES)
    return jnp.transpose(out4, (1, 3, 0, 2)).reshape(n_batch, n_seq, DIM)
